# baseline (speedup 1.0000x reference)
import functools
import jax
import jax.numpy as jnp
from jax import lax
from jax.experimental import pallas as pl
from jax.experimental.pallas import tpu as pltpu
from jax.experimental.pallas import tpu_sc as plsc

_SIZE = 26
_D = 2048
_LEVELS = 100
_LP = 104
_K = _SIZE * _LP
_BATCH = 1024
_ZROW = 103


def _build_w_body(pos_ref, lev_ref, w_ref):
    lev = lev_ref[...]
    rows = lax.broadcasted_iota(jnp.int32, (_LP, _D), 0)
    lev = jnp.where(rows < _LEVELS, lev, 0.0)
    for s in range(_SIZE):
        p = pos_ref[s:s + 1, :]
        w_ref[s * _LP:(s + 1) * _LP, :] = lev * p


def _build_w(position_weight, level_weight):
    return pl.pallas_call(
        _build_w_body,
        grid=(1,),
        in_specs=[pl.BlockSpec((32, _D), lambda i: (0, 0)),
                  pl.BlockSpec((_LP, _D), lambda i: (0, 0))],
        out_specs=pl.BlockSpec((_K, _D), lambda i: (0, 0)),
        out_shape=jax.ShapeDtypeStruct((_K, _D), jnp.float32),
    )(position_weight, level_weight)


_NC = 2
_NS = 16
_NW = _NC * _NS
_BPW = _BATCH // _NW
_FL = _BPW * _SIZE


def _sc_kernel(w_hbm, xf_hbm, off_hbm, out_hbm,
               xv, offv, fidv, rows_lo, rows_hi, outv, sem):
    wid = lax.axis_index("s") * _NC + lax.axis_index("c")
    base = wid * _FL
    pltpu.sync_copy(xf_hbm.at[pl.ds(base, _FL)], xv)
    pltpu.sync_copy(off_hbm.at[pl.ds(base, _FL)], offv)

    for k in range(_FL // 16):
        x16 = xv[pl.ds(k * 16, 16)]
        r = x16 * 99.0
        n = r.astype(jnp.int32)
        d = r - n.astype(jnp.float32)
        one = jnp.full((16,), 1, jnp.int32)
        zero = jnp.full((16,), 0, jnp.int32)
        inc = jnp.where(d > 0.5, one, zero)
        tie = jnp.where((d == 0.5) & ((n & 1) == 1), one, zero)
        idx16 = jnp.clip(n + inc + tie, 0, _LEVELS - 1)
        fidv[pl.ds(k * 16, 16)] = idx16 + offv[pl.ds(k * 16, 16)]

    lanes = lax.iota(jnp.int32, 16)

    def body_b(b, carry):
        lo16 = fidv[pl.ds(b * _SIZE, 16)]
        hi_raw = fidv[pl.ds(b * _SIZE + 16, 16)]
        hi16 = jnp.where(lanes < _SIZE - 16, hi_raw, _ZROW)
        pltpu.async_copy(w_hbm.at[lo16], rows_lo, sem).wait()
        pltpu.async_copy(w_hbm.at[hi16], rows_hi, sem).wait()

        def body_j(j, c):
            acc = rows_lo[0, pl.ds(j * 16, 16)]
            for r in range(1, 16):
                acc = acc + rows_lo[r, pl.ds(j * 16, 16)]
            for r in range(16):
                acc = acc + rows_hi[r, pl.ds(j * 16, 16)]
            outv[pl.ds(j * 16, 16)] = jnp.where(acc > 0.0, 1.0, -1.0)
            return c

        lax.fori_loop(0, _D // 16, body_j, 0)
        pltpu.sync_copy(outv, out_hbm.at[wid * _BPW + b])
        return carry

    lax.fori_loop(0, _BPW, body_b, 0)


def kernel(x, position_weight, level_weight):
    w = _build_w(position_weight, level_weight)
    xf = x.reshape(-1)
    offs = jnp.tile(jnp.arange(_SIZE, dtype=jnp.int32) * _LP, _BATCH)
    mesh = plsc.VectorSubcoreMesh(core_axis_name="c", subcore_axis_name="s")
    f = functools.partial(
        pl.kernel, mesh=mesh,
        out_type=jax.ShapeDtypeStruct((_BATCH, _D), jnp.float32),
        scratch_types=[
            pltpu.VMEM((_FL,), jnp.float32),
            pltpu.VMEM((_FL,), jnp.int32),
            pltpu.VMEM((_FL + 16,), jnp.int32),
            pltpu.VMEM((16, _D), jnp.float32),
            pltpu.VMEM((16, _D), jnp.float32),
            pltpu.VMEM((_D,), jnp.float32),
            pltpu.SemaphoreType.DMA,
        ],
    )(_sc_kernel)
    return f(w, xf, offs)


# speedup vs baseline: 28.1116x; 28.1116x over previous
import jax
import jax.numpy as jnp
from jax.experimental import pallas as pl
from jax.experimental.pallas import tpu as pltpu

_SIZE = 26
_D = 2048
_LEVELS = 100
_LP = 104
_K = _SIZE * _LP
_BATCH = 1024
_BB = 256


def _body(x_ref, pos_ref, lev_ref, out_ref, w_ref, e_ref, lmod_ref):
    i = pl.program_id(0)

    @pl.when(i == 0)
    def _build_tables():
        lev = lev_ref[...]
        rows = jax.lax.broadcasted_iota(jnp.int32, (_LP, _D), 0)
        lev = jnp.where(rows < _LEVELS, lev, 0.0)
        for s in range(_SIZE):
            p = pos_ref[s:s + 1, :]
            w_ref[s * _LP:(s + 1) * _LP, :] = (lev * p).astype(jnp.bfloat16)
        js = jax.lax.broadcasted_iota(jnp.int32, (32, _K), 1) // _LP
        ss = jax.lax.broadcasted_iota(jnp.int32, (32, _K), 0)
        e_ref[...] = jnp.where(js == ss, 1.0, 0.0).astype(jnp.bfloat16)
        cols = jax.lax.broadcasted_iota(jnp.int32, (_BB, _K), 1)
        lmod_ref[...] = (cols % _LP).astype(jnp.bfloat16)

    xb = x_ref[...]
    idx = jnp.clip(jnp.round(xb * (_LEVELS - 1)).astype(jnp.int32),
                   0, _LEVELS - 1)
    idx32 = jnp.pad(idx.astype(jnp.bfloat16), ((0, 0), (0, 32 - _SIZE)))
    expand = jnp.dot(idx32, e_ref[...],
                     preferred_element_type=jnp.float32)
    oh = (expand.astype(jnp.bfloat16) == lmod_ref[...]).astype(jnp.bfloat16)
    acc = jnp.dot(oh, w_ref[...], preferred_element_type=jnp.float32)
    out_ref[...] = jnp.where(acc > 0, 1.0, -1.0).astype(jnp.float32)


def kernel(x, position_weight, level_weight):
    return pl.pallas_call(
        _body,
        grid=(_BATCH // _BB,),
        in_specs=[
            pl.BlockSpec((_BB, _SIZE), lambda i: (i, 0)),
            pl.BlockSpec((32, _D), lambda i: (0, 0)),
            pl.BlockSpec((_LP, _D), lambda i: (0, 0)),
        ],
        out_specs=pl.BlockSpec((_BB, _D), lambda i: (i, 0)),
        out_shape=jax.ShapeDtypeStruct((_BATCH, _D), jnp.float32),
        scratch_shapes=[pltpu.VMEM((_K, _D), jnp.bfloat16),
                        pltpu.VMEM((32, _K), jnp.bfloat16),
                        pltpu.VMEM((_BB, _K), jnp.bfloat16)],
    )(x, position_weight, level_weight)


# speedup vs baseline: 28.7350x; 1.0222x over previous
import jax
import jax.numpy as jnp
from jax.experimental import pallas as pl
from jax.experimental.pallas import tpu as pltpu

_SIZE = 26
_D = 2048
_LEVELS = 100
_LP = 104
_K = _SIZE * _LP
_BATCH = 1024
_BB = 256


def _body(x_ref, pos_ref, lev_ref, out_ref, w_ref, e_ref, lmod_ref):
    i = pl.program_id(0)

    @pl.when(i == 0)
    def _build_tables():
        lev = lev_ref[...]
        rows = jax.lax.broadcasted_iota(jnp.int32, (_LP, _D), 0)
        lev = jnp.where(rows < _LEVELS, lev, 0.0)
        for s in range(_SIZE):
            p = pos_ref[s:s + 1, :]
            w_ref[s * _LP:(s + 1) * _LP, :] = (lev * p).astype(jnp.bfloat16)
        js = jax.lax.broadcasted_iota(jnp.int32, (32, _K), 1) // _LP
        ss = jax.lax.broadcasted_iota(jnp.int32, (32, _K), 0)
        e_ref[...] = jnp.where(js == ss, 1.0, 0.0).astype(jnp.bfloat16)
        cols = jax.lax.broadcasted_iota(jnp.int32, (_BB, _K), 1)
        lmod_ref[...] = (cols % _LP).astype(jnp.float32)

    xb = x_ref[...]
    idx = jnp.clip(jnp.round(xb * (_LEVELS - 1)).astype(jnp.int32),
                   0, _LEVELS - 1)
    idx32 = jnp.pad(idx.astype(jnp.bfloat16), ((0, 0), (0, 32 - _SIZE)))
    expand = jnp.dot(idx32, e_ref[...],
                     preferred_element_type=jnp.float32)
    oh = (expand == lmod_ref[...]).astype(jnp.bfloat16)
    acc = jnp.dot(oh, w_ref[...], preferred_element_type=jnp.float32)
    out_ref[...] = jnp.where(acc > 0, 1.0, -1.0).astype(jnp.float32)


def kernel(x, position_weight, level_weight):
    return pl.pallas_call(
        _body,
        grid=(_BATCH // _BB,),
        in_specs=[
            pl.BlockSpec((_BB, _SIZE), lambda i: (i, 0)),
            pl.BlockSpec((32, _D), lambda i: (0, 0)),
            pl.BlockSpec((_LP, _D), lambda i: (0, 0)),
        ],
        out_specs=pl.BlockSpec((_BB, _D), lambda i: (i, 0)),
        out_shape=jax.ShapeDtypeStruct((_BATCH, _D), jnp.float32),
        scratch_shapes=[pltpu.VMEM((_K, _D), jnp.bfloat16),
                        pltpu.VMEM((32, _K), jnp.bfloat16),
                        pltpu.VMEM((_BB, _K), jnp.float32)],
    )(x, position_weight, level_weight)
